# speedup vs baseline: 1281.9768x; 1281.9768x over previous
import jax
import jax.numpy as jnp
from jax.experimental import pallas as pl

_B = 128
_FEATURES = [256, 64, 32, 3]
_BRANCHES = [4, 8, 16]
_NL = [1, 4, 32, 512]
_N_COND = 1
_N_GLOBAL = 8


def _mm(Wt, X):
    d, n, b = X.shape
    y = jax.lax.dot_general(Wt, X.reshape(d, n * b),
                            (((1,), (0,)), ((), ())),
                            preferred_element_type=jnp.float32)
    return y.reshape(Wt.shape[0], n, b)


def _body(random_vector, cond, W_hlv, b_hlv, W_br, W_red, W_amsg, W_aupd,
          W_cmsg, W_cupd, scale, out_ref):
    cond_t = cond[...].T
    X0 = jnp.concatenate([cond_t, random_vector[...].T[_N_COND:]], axis=0)
    X = [X0[:, None, :]]

    for il in range(3):
        d_in, d_out, br = _FEATURES[il], _FEATURES[il + 1], _BRANCHES[il]
        n = _NL[il]
        Xl = X[il]
        pooled = Xl[:_FEATURES[-1]].mean(axis=1)
        hlv_in = jnp.concatenate([pooled, cond_t], axis=0)
        glob = jax.nn.relu(W_hlv[il][...].T @ hlv_in + b_hlv[il][...].T)

        feats = jnp.concatenate([
            Xl,
            jnp.broadcast_to(cond_t[:, None, :], (_N_COND, n, _B)),
            jnp.broadcast_to(glob[:, None, :], (_N_GLOBAL, n, _B)),
        ], axis=0)
        ch = jax.nn.relu(_mm(W_br[il][...].T, feats))
        ch = ch.reshape(br, d_out, n, _B).transpose(1, 2, 0, 3).reshape(d_out, n * br, _B)

        Wred_t = W_red[il][...].T
        X = [_mm(Wred_t, Xk) for Xk in X]
        X.append(ch)
        L = il + 1
        nL = _NL[L]

        Wam = W_amsg[il][...]
        Wa_src = Wam[:d_out].T
        Wa_dst = Wam[d_out:2 * d_out].T
        w_ea = Wam[2 * d_out]
        Wa_cg = Wam[2 * d_out + 1:].T
        cg = jnp.concatenate([cond_t, glob], axis=0)
        CG = Wa_cg @ cg
        D = _mm(Wa_dst, X[L]) + CG[:, None, :]
        agg = jnp.zeros((d_out, nL, _B), jnp.float32)
        for a in range(L):
            stride = 1
            for bb in _BRANCHES[a:L - 1 + 1]:
                stride *= bb
            S = _mm(Wa_src, X[a])
            Sb = jnp.broadcast_to(S[:, :, None, :], (d_out, _NL[a], stride, _B))
            Sb = Sb.reshape(d_out, nL, _B)
            ea = float(L - a)
            agg = agg + jax.nn.relu(Sb + D + ea * w_ea[:, None, None])

        Wau = W_aupd[il][...]
        Wau_t, Wau_b = Wau[:d_out].T, Wau[d_out:].T
        X = [jax.nn.relu(_mm(Wau_t, X[k]) + (_mm(Wau_b, agg) if k == L else 0.0))
             for k in range(L + 1)]

        Wcm = W_cmsg[il][...]
        Wc_src = Wcm[:d_out].T
        Wc_dst = Wcm[d_out:2 * d_out].T
        Wc_cg = Wcm[2 * d_out:].T
        CGc = Wc_cg @ cg
        ng = nL // br
        A = _mm(Wc_src, X[L]).reshape(d_out, ng, br, _B)
        Bv = (_mm(Wc_dst, X[L]) + CGc[:, None, :]).reshape(d_out, ng, br, _B)
        cagg = jnp.zeros_like(Bv)
        for i in range(br):
            cagg = cagg + jax.nn.relu(A[:, :, i:i + 1, :] + Bv)
        cagg = cagg.reshape(d_out, nL, _B)

        Wcu = W_cupd[il][...]
        Wcu_t, Wcu_b = Wcu[:d_out].T, Wcu[d_out:].T
        X = [jax.nn.relu(_mm(Wcu_t, X[k]) + (_mm(Wcu_b, cagg) if k == L else 0.0))
             for k in range(L + 1)]

    X3 = X[3] * scale[...].T[:, :, None]
    out_ref[...] = X3.transpose(0, 2, 1).reshape(3, _B * _NL[3])


def _pallas_body(rv, cond,
                 hlv0, bh0, br0, red0, am0, au0, cm0, cu0,
                 hlv1, bh1, br1, red1, am1, au1, cm1, cu1,
                 hlv2, bh2, br2, red2, am2, au2, cm2, cu2,
                 scale, out_ref):
    _body(rv, cond,
          [hlv0, hlv1, hlv2], [bh0, bh1, bh2], [br0, br1, br2],
          [red0, red1, red2], [am0, am1, am2], [au0, au1, au2],
          [cm0, cm1, cm2], [cu0, cu1, cu2], scale, out_ref)


def kernel(random_vector, cond,
           W_hlv_0, b_hlv_0, W_br_0, W_red_0, W_amsg_0, W_aupd_0, W_cmsg_0, W_cupd_0,
           W_hlv_1, b_hlv_1, W_br_1, W_red_1, W_amsg_1, W_aupd_1, W_cmsg_1, W_cupd_1,
           W_hlv_2, b_hlv_2, W_br_2, W_red_2, W_amsg_2, W_aupd_2, W_cmsg_2, W_cupd_2,
           scale):
    operands = (
        random_vector, cond,
        W_hlv_0, b_hlv_0.reshape(1, _N_GLOBAL), W_br_0, W_red_0, W_amsg_0,
        W_aupd_0, W_cmsg_0, W_cupd_0,
        W_hlv_1, b_hlv_1.reshape(1, _N_GLOBAL), W_br_1, W_red_1, W_amsg_1,
        W_aupd_1, W_cmsg_1, W_cupd_1,
        W_hlv_2, b_hlv_2.reshape(1, _N_GLOBAL), W_br_2, W_red_2, W_amsg_2,
        W_aupd_2, W_cmsg_2, W_cupd_2,
        scale.reshape(1, _FEATURES[-1]),
    )
    out = pl.pallas_call(
        _pallas_body,
        out_shape=jax.ShapeDtypeStruct((_FEATURES[-1], _B * _NL[3]), jnp.float32),
    )(*operands)
    return out.T
